# baseline (speedup 1.0000x reference)
import jax
import jax.numpy as jnp
from jax.experimental import pallas as pl

TOP_K = 8
N_EXPERTS = 64
N_GROUP = 8
GROUP_SIZE = 8
TOPK_GROUP = 4
SCALE = 2.5

BT = 2048


def _router_block(x_ref, wt_ref, w_ref_in, biast_ref, logits_ref, idx_ref,
                  w_ref):
    x = x_ref[...]
    logits_t = jax.lax.dot_general(
        w_ref_in[...], x, (((1,), (1,)), ((), ())),
        preferred_element_type=jnp.float32)
    logits_ref[...] = jnp.transpose(logits_t)

    bt = x.shape[0]
    scores = jax.nn.sigmoid(logits_t)
    s4c = scores + biast_ref[...]
    row = jax.lax.broadcasted_iota(jnp.int32, (N_EXPERTS, bt), 0)
    rowf = row.astype(jnp.float32)
    gid = row // GROUP_SIZE
    neg = jnp.float32(-jnp.inf)
    one = jnp.float32(1.0)

    def partner(v, d):
        return jnp.where((row & d) == 0, jnp.roll(v, -d, axis=0),
                         jnp.roll(v, d, axis=0))

    p = partner(s4c, 1)
    hi = jnp.maximum(s4c, p)
    lo = jnp.minimum(s4c, p)
    for d in (2, 4):
        ph = partner(hi, d)
        plo = jnp.where(hi >= ph, lo, partner(lo, d))
        hi, lo = jnp.maximum(hi, ph), jnp.maximum(jnp.minimum(hi, ph), plo)
    gs = hi + lo

    rankf = jnp.zeros((N_EXPERTS, bt), jnp.float32)
    for k in (1, 2, 3, 4):
        r = jnp.roll(gs, GROUP_SIZE * k, axis=0)
        bk = jnp.where((r > gs) | ((r == gs) & (gid >= k)), one, 0.0)
        rankf = rankf + bk
        if k < 4:
            rankf = rankf + (one - jnp.roll(bk, -GROUP_SIZE * k, axis=0))
    mask = rankf < TOPK_GROUP

    cur = jnp.where(mask, s4c, 0.0)
    kc = rowf * 2.0 - scores
    big = jnp.float32(200.0)
    kmins = []
    for _ in range(TOP_K):
        vmax = jnp.max(cur, axis=0, keepdims=True)
        kmin = jnp.min(jnp.where(cur == vmax, kc, big), axis=0, keepdims=True)
        kmins.append(kmin)
        m2 = kc == kmin
        cur = jnp.where(m2, neg, cur)
    kacc = jnp.concatenate(kmins, axis=0)
    sel_f = jnp.ceil(kacc * 0.5)
    w_all = 2.0 * sel_f - kacc
    denom = jnp.sum(w_all, axis=0, keepdims=True) + 1e-20
    w_out = w_all * (SCALE / denom)
    idx_ref[...] = jnp.transpose(sel_f.astype(jnp.int32))
    w_ref[...] = jnp.transpose(w_out)


@jax.jit
def kernel(hidden_states, weight, e_score_correction_bias):
    b, s, h = hidden_states.shape
    t = b * s
    hs = hidden_states.reshape(t, h).astype(jnp.float32)
    w = weight.astype(jnp.float32)
    wt = w.T
    bias_t = e_score_correction_bias.astype(jnp.float32).reshape(N_EXPERTS, 1)

    grid = (t // BT,)
    logits, idx, wts = pl.pallas_call(
        _router_block,
        grid=grid,
        in_specs=[
            pl.BlockSpec((BT, h), lambda i: (i, 0)),
            pl.BlockSpec((h, N_EXPERTS), lambda i: (0, 0)),
            pl.BlockSpec((N_EXPERTS, h), lambda i: (0, 0)),
            pl.BlockSpec((N_EXPERTS, 1), lambda i: (0, 0)),
        ],
        out_specs=[
            pl.BlockSpec((BT, N_EXPERTS), lambda i: (i, 0)),
            pl.BlockSpec((BT, TOP_K), lambda i: (i, 0)),
            pl.BlockSpec((BT, TOP_K), lambda i: (i, 0)),
        ],
        out_shape=[
            jax.ShapeDtypeStruct((t, N_EXPERTS), jnp.float32),
            jax.ShapeDtypeStruct((t, TOP_K), jnp.int32),
            jax.ShapeDtypeStruct((t, TOP_K), jnp.float32),
        ],
    )(hs, wt, w, bias_t)
    return idx, wts, logits


# speedup vs baseline: 1.0417x; 1.0417x over previous
import jax
import jax.numpy as jnp
from jax.experimental import pallas as pl

TOP_K = 8
N_EXPERTS = 64
N_GROUP = 8
GROUP_SIZE = 8
TOPK_GROUP = 4
SCALE = 2.5

BT = 4096


def _router_block(x_ref, wt_ref, w_ref_in, biast_ref, logits_ref, idx_ref,
                  w_ref):
    x = x_ref[...]
    logits_t = jax.lax.dot_general(
        w_ref_in[...], x, (((1,), (1,)), ((), ())),
        preferred_element_type=jnp.float32)
    logits_ref[...] = jnp.transpose(logits_t)

    bt = x.shape[0]
    scores = jax.nn.sigmoid(logits_t)
    s4c = scores + biast_ref[...]
    row = jax.lax.broadcasted_iota(jnp.int32, (N_EXPERTS, bt), 0)
    rowf = row.astype(jnp.float32)
    gid = row // GROUP_SIZE
    neg = jnp.float32(-jnp.inf)
    one = jnp.float32(1.0)

    def partner(v, d):
        return jnp.where((row & d) == 0, jnp.roll(v, -d, axis=0),
                         jnp.roll(v, d, axis=0))

    p = partner(s4c, 1)
    hi = jnp.maximum(s4c, p)
    lo = jnp.minimum(s4c, p)
    for d in (2, 4):
        ph = partner(hi, d)
        plo = jnp.where(hi >= ph, lo, partner(lo, d))
        hi, lo = jnp.maximum(hi, ph), jnp.maximum(jnp.minimum(hi, ph), plo)
    gs = hi + lo

    rankf = jnp.zeros((N_EXPERTS, bt), jnp.float32)
    for k in (1, 2, 3, 4):
        r = jnp.roll(gs, GROUP_SIZE * k, axis=0)
        bk = jnp.where((r > gs) | ((r == gs) & (gid >= k)), one, 0.0)
        rankf = rankf + bk
        if k < 4:
            rankf = rankf + (one - jnp.roll(bk, -GROUP_SIZE * k, axis=0))
    mask = rankf < TOPK_GROUP

    cur = jnp.where(mask, s4c, 0.0)
    kc = rowf * 2.0 - scores
    big = jnp.float32(200.0)
    kmins = []
    for _ in range(TOP_K):
        vmax = jnp.max(cur, axis=0, keepdims=True)
        kmin = jnp.min(jnp.where(cur == vmax, kc, big), axis=0, keepdims=True)
        kmins.append(kmin)
        m2 = kc == kmin
        cur = jnp.where(m2, neg, cur)
    kacc = jnp.concatenate(kmins, axis=0)
    sel_f = jnp.ceil(kacc * 0.5)
    w_all = 2.0 * sel_f - kacc
    denom = jnp.sum(w_all, axis=0, keepdims=True) + 1e-20
    w_out = w_all * (SCALE / denom)
    idx_ref[...] = jnp.transpose(sel_f.astype(jnp.int32))
    w_ref[...] = jnp.transpose(w_out)


@jax.jit
def kernel(hidden_states, weight, e_score_correction_bias):
    b, s, h = hidden_states.shape
    t = b * s
    hs = hidden_states.reshape(t, h).astype(jnp.float32)
    w = weight.astype(jnp.float32)
    wt = w.T
    bias_t = e_score_correction_bias.astype(jnp.float32).reshape(N_EXPERTS, 1)

    grid = (t // BT,)
    logits, idx, wts = pl.pallas_call(
        _router_block,
        grid=grid,
        in_specs=[
            pl.BlockSpec((BT, h), lambda i: (i, 0)),
            pl.BlockSpec((h, N_EXPERTS), lambda i: (0, 0)),
            pl.BlockSpec((N_EXPERTS, h), lambda i: (0, 0)),
            pl.BlockSpec((N_EXPERTS, 1), lambda i: (0, 0)),
        ],
        out_specs=[
            pl.BlockSpec((BT, N_EXPERTS), lambda i: (i, 0)),
            pl.BlockSpec((BT, TOP_K), lambda i: (i, 0)),
            pl.BlockSpec((BT, TOP_K), lambda i: (i, 0)),
        ],
        out_shape=[
            jax.ShapeDtypeStruct((t, N_EXPERTS), jnp.float32),
            jax.ShapeDtypeStruct((t, TOP_K), jnp.int32),
            jax.ShapeDtypeStruct((t, TOP_K), jnp.float32),
        ],
    )(hs, wt, w, bias_t)
    return idx, wts, logits
